# baseline (speedup 1.0000x reference)
import functools

import jax
import jax.numpy as jnp
from jax import lax
from jax.experimental import pallas as pl
from jax.experimental.pallas import tpu as pltpu
from jax.experimental.pallas import tpu_sc as plsc

N_NODES = 10000
N_EDGES = 320000
D_FEAT = 128
D_EDGE = 16
HID = 128
EXW = 16

NC, NS = 2, 16
NW = NC * NS
EPW = N_EDGES // NW
GCH = 80
NCH = EPW // GCH
N_PAD = 10240
NPS = N_PAD // NS

_SC_PARAMS = pltpu.CompilerParams(use_tc_tiling_on_sc=False)


@functools.cache
def _mesh():
    return plsc.VectorSubcoreMesh(
        core_axis_name="c", subcore_axis_name="s",
        num_cores=NC, num_subcores=NS)


def _prelude_body(nodes_ref, wsq_ref, p_ref, q_ref):
    pq = jnp.dot(nodes_ref[...], wsq_ref[...],
                 preferred_element_type=jnp.float32)
    p_ref[...] = pq[:, :HID].astype(jnp.bfloat16)
    q_ref[...] = pq[:, HID:].astype(jnp.bfloat16)


def _msg_body(e_ref, sr_ref, w1e_ref, b1_ref, w2_ref, b2_ref,
              wg1_ref, bg1_ref, wg2p_ref, bg2_ref, m_ref, x_ref):
    h = sr_ref[...].astype(jnp.float32) + jnp.dot(e_ref[...], w1e_ref[...],
                              preferred_element_type=jnp.float32) + b1_ref[...]
    h = jnp.maximum(h, 0.0)
    msgs = jnp.dot(h, w2_ref[...], preferred_element_type=jnp.float32) + b2_ref[...]
    g = jnp.maximum(jnp.dot(msgs, wg1_ref[...],
                            preferred_element_type=jnp.float32) + bg1_ref[...], 0.0)
    gate = jnp.dot(g, wg2p_ref[...],
                   preferred_element_type=jnp.float32) + bg2_ref[...]
    ex16 = jnp.exp(gate)
    m_ref[...] = ex16[:, 0:1] * msgs
    x_ref[...] = ex16


def _upd_body(nodes_ref, am_ref, ae_ref, wu1a_ref, wu1b_ref, bu1_ref,
              wu2_ref, bu2_ref, out_ref):
    am = am_ref[...]
    num = am[0] + am[1]
    ae = ae_ref[...]
    den = (ae[0] + ae[1])[:, 0:1]
    aggr = jnp.where(den > 0.0, num / den, 0.0)
    u = (jnp.dot(nodes_ref[...], wu1a_ref[...], preferred_element_type=jnp.float32)
         + jnp.dot(aggr, wu1b_ref[...], preferred_element_type=jnp.float32)
         + bu1_ref[...])
    u = jnp.maximum(u, 0.0)
    out_ref[...] = jnp.dot(u, wu2_ref[...],
                           preferred_element_type=jnp.float32) + bu2_ref[...]


@functools.cache
def _gather_kernel():
    return pl.kernel(
        _gather_body,
        out_type=jax.ShapeDtypeStruct((N_EDGES, HID), jnp.bfloat16),
        mesh=_mesh(),
        compiler_params=_SC_PARAMS,
        scratch_types=(
            [pltpu.VMEM((NCH, GCH), jnp.int32)] * 2
            + [pltpu.VMEM((GCH, HID), jnp.bfloat16)] * 4
            + [pltpu.SemaphoreType.DMA] * 4
        ),
    )


def _gather_body(p_hbm, q_hbm, s3_hbm, r3_hbm, out_hbm,
                 sidx2, ridx2, pb0, qb0, pb1, qb1, gs0, gs1, ws0, ws1):
    wid = lax.axis_index("s") * NC + lax.axis_index("c")
    base0 = wid * EPW
    pltpu.sync_copy(s3_hbm.at[wid], sidx2)
    pltpu.sync_copy(r3_hbm.at[wid], ridx2)

    def issue(ci, pb, qb, gs):
        pltpu.async_copy(p_hbm.at[sidx2.at[ci]], pb, gs)
        pltpu.async_copy(q_hbm.at[ridx2.at[ci]], qb, gs)

    def gwait(pb, qb, gs):
        pltpu.make_async_copy(p_hbm.at[sidx2.at[0]], pb, gs).wait()
        pltpu.make_async_copy(q_hbm.at[ridx2.at[0]], qb, gs).wait()

    def add_rows(pb, qb):
        @pl.loop(0, GCH)
        def _r(e):
            for c in range(0, HID, 32):
                pb[e, pl.ds(c, 32)] = pb[e, pl.ds(c, 32)] + qb[e, pl.ds(c, 32)]

    def wstart(ci, pb, ws):
        pltpu.async_copy(pb, out_hbm.at[pl.ds(base0 + ci * GCH, GCH)], ws)

    def wwait(pb, ws):
        pltpu.make_async_copy(pb, out_hbm.at[pl.ds(base0, GCH)], ws).wait()

    issue(0, pb0, qb0, gs0)
    issue(1, pb1, qb1, gs1)

    @pl.loop(0, (NCH - 1) // 2)
    def _k(k):
        gwait(pb0, qb0, gs0)
        add_rows(pb0, qb0)
        wstart(2 * k, pb0, ws0)
        wwait(pb0, ws0)
        issue(2 * k + 2, pb0, qb0, gs0)

        gwait(pb1, qb1, gs1)
        add_rows(pb1, qb1)
        wstart(2 * k + 1, pb1, ws1)
        wwait(pb1, ws1)
        issue(jnp.minimum(2 * k + 3, NCH - 1), pb1, qb1, gs1)

    gwait(pb0, qb0, gs0)
    add_rows(pb0, qb0)
    wstart(NCH - 1, pb0, ws0)
    wwait(pb0, ws0)
    gwait(pb1, qb1, gs1)


@functools.cache
def _scatter_kernel():
    return pl.kernel(
        _scatter_body,
        out_type=[
            jax.ShapeDtypeStruct((NC, N_PAD, HID), jnp.float32),
            jax.ShapeDtypeStruct((NC, N_PAD, EXW), jnp.float32),
        ],
        mesh=_mesh(),
        compiler_params=_SC_PARAMS,
        scratch_types=[
            pltpu.VMEM((NCH, GCH), jnp.int32),
            pltpu.VMEM((GCH, HID), jnp.float32),
            pltpu.VMEM((GCH, EXW), jnp.float32),
            pltpu.VMEM((GCH, HID), jnp.float32),
            pltpu.VMEM((GCH, EXW), jnp.float32),
            pltpu.VMEM_SHARED((N_PAD, HID), jnp.float32),
            pltpu.VMEM_SHARED((N_PAD, EXW), jnp.float32),
            pltpu.SemaphoreType.DMA,
            pltpu.SemaphoreType.DMA,
            pltpu.SemaphoreType.DMA,
            pltpu.SemaphoreType.DMA,
        ],
    )


def _scatter_body(exm_hbm, ex_hbm, r3_hbm, om_hbm, oe_hbm,
                  ridx2, mb0, eb0, mb1, eb1, accm, acce,
                  ls0, ls1, ss0, ss1):
    cid = lax.axis_index("c")
    sid = lax.axis_index("s")
    wid = sid * NC + cid
    base0 = wid * EPW

    @pl.loop(0, GCH)
    def _z(i):
        for c in range(0, HID, 16):
            mb0[i, pl.ds(c, 16)] = jnp.zeros((16,), jnp.float32)
        eb0[i, pl.ds(0, EXW)] = jnp.zeros((EXW,), jnp.float32)

    @pl.loop(0, NPS // GCH)
    def _zc(j):
        pltpu.sync_copy(mb0, accm.at[pl.ds(sid * NPS + j * GCH, GCH)])
        pltpu.sync_copy(eb0, acce.at[pl.ds(sid * NPS + j * GCH, GCH)])

    pltpu.sync_copy(r3_hbm.at[wid], ridx2)
    plsc.subcore_barrier()

    def lissue(ci, mb, eb, ls):
        pltpu.async_copy(exm_hbm.at[pl.ds(base0 + ci * GCH, GCH)], mb, ls)
        pltpu.async_copy(ex_hbm.at[pl.ds(base0 + ci * GCH, GCH)], eb, ls)

    def lwait(mb, eb, ls):
        pltpu.make_async_copy(exm_hbm.at[pl.ds(base0, GCH)], mb, ls).wait()
        pltpu.make_async_copy(ex_hbm.at[pl.ds(base0, GCH)], eb, ls).wait()

    def sstart(ci, mb, eb, ss):
        pltpu.async_copy(mb, accm.at[ridx2.at[ci]], ss, add=True)
        pltpu.async_copy(eb, acce.at[ridx2.at[ci]], ss, add=True)

    def swait(ci, mb, eb, ss):
        pltpu.make_async_copy(mb, accm.at[ridx2.at[ci]], ss).wait()
        pltpu.make_async_copy(eb, acce.at[ridx2.at[ci]], ss).wait()

    lissue(0, mb0, eb0, ls0)
    lissue(1, mb1, eb1, ls1)

    @pl.loop(0, (NCH - 1) // 2)
    def _k(k):
        lwait(mb0, eb0, ls0)
        sstart(2 * k, mb0, eb0, ss0)
        swait(2 * k, mb0, eb0, ss0)
        lissue(2 * k + 2, mb0, eb0, ls0)

        lwait(mb1, eb1, ls1)
        sstart(2 * k + 1, mb1, eb1, ss1)
        swait(2 * k + 1, mb1, eb1, ss1)
        lissue(jnp.minimum(2 * k + 3, NCH - 1), mb1, eb1, ls1)

    lwait(mb0, eb0, ls0)
    sstart(NCH - 1, mb0, eb0, ss0)
    swait(NCH - 1, mb0, eb0, ss0)
    lwait(mb1, eb1, ls1)

    plsc.subcore_barrier()
    pltpu.sync_copy(accm.at[pl.ds(sid * NPS, NPS)],
                    om_hbm.at[cid, pl.ds(sid * NPS, NPS)])
    pltpu.sync_copy(acce.at[pl.ds(sid * NPS, NPS)],
                    oe_hbm.at[cid, pl.ds(sid * NPS, NPS)])


def kernel(nodes, edges, W1, b1, W2, b2, Wg1, bg1, Wg2, bg2,
           Wu1, bu1, Wu2, bu2, senders, receivers):
    f32 = jnp.float32
    s3 = senders.astype(jnp.int32).reshape(NW, NCH, GCH)
    r3 = receivers.astype(jnp.int32).reshape(NW, NCH, GCH)

    wsq = W1[D_EDGE:, :]
    wsq = jnp.concatenate([wsq[:D_FEAT], wsq[D_FEAT:]], axis=1)
    w1e = W1[:D_EDGE, :]
    wg2p = jnp.tile(Wg2, (1, EXW))

    NB = 2000
    p, q = pl.pallas_call(
        _prelude_body,
        grid=(N_NODES // NB,),
        in_specs=[
            pl.BlockSpec((NB, D_FEAT), lambda i: (i, 0)),
            pl.BlockSpec((D_FEAT, 2 * HID), lambda i: (0, 0)),
        ],
        out_specs=[
            pl.BlockSpec((NB, HID), lambda i: (i, 0)),
            pl.BlockSpec((NB, HID), lambda i: (i, 0)),
        ],
        out_shape=[
            jax.ShapeDtypeStruct((N_NODES, HID), jnp.bfloat16),
            jax.ShapeDtypeStruct((N_NODES, HID), jnp.bfloat16),
        ],
    )(nodes, wsq)

    sr = _gather_kernel()(p, q, s3, r3)

    EB = 2000
    exm, ex = pl.pallas_call(
        _msg_body,
        grid=(N_EDGES // EB,),
        in_specs=[
            pl.BlockSpec((EB, D_EDGE), lambda i: (i, 0)),
            pl.BlockSpec((EB, HID), lambda i: (i, 0)),
            pl.BlockSpec((D_EDGE, HID), lambda i: (0, 0)),
            pl.BlockSpec((1, HID), lambda i: (0, 0)),
            pl.BlockSpec((HID, HID), lambda i: (0, 0)),
            pl.BlockSpec((1, HID), lambda i: (0, 0)),
            pl.BlockSpec((HID, HID), lambda i: (0, 0)),
            pl.BlockSpec((1, HID), lambda i: (0, 0)),
            pl.BlockSpec((HID, EXW), lambda i: (0, 0)),
            pl.BlockSpec((1, 1), lambda i: (0, 0)),
        ],
        out_specs=[
            pl.BlockSpec((EB, HID), lambda i: (i, 0)),
            pl.BlockSpec((EB, EXW), lambda i: (i, 0)),
        ],
        out_shape=[
            jax.ShapeDtypeStruct((N_EDGES, HID), f32),
            jax.ShapeDtypeStruct((N_EDGES, EXW), f32),
        ],
    )(edges, sr, w1e, b1.reshape(1, HID), W2, b2.reshape(1, HID),
      Wg1, bg1.reshape(1, HID), wg2p, bg2.reshape(1, 1))

    parts_m, parts_e = _scatter_kernel()(exm, ex, r3)

    UB = 2000
    out = pl.pallas_call(
        _upd_body,
        grid=(N_NODES // UB,),
        in_specs=[
            pl.BlockSpec((UB, D_FEAT), lambda i: (i, 0)),
            pl.BlockSpec((NC, UB, HID), lambda i: (0, i, 0)),
            pl.BlockSpec((NC, UB, EXW), lambda i: (0, i, 0)),
            pl.BlockSpec((D_FEAT, HID), lambda i: (0, 0)),
            pl.BlockSpec((HID, HID), lambda i: (0, 0)),
            pl.BlockSpec((1, HID), lambda i: (0, 0)),
            pl.BlockSpec((HID, HID), lambda i: (0, 0)),
            pl.BlockSpec((1, HID), lambda i: (0, 0)),
        ],
        out_specs=pl.BlockSpec((UB, HID), lambda i: (i, 0)),
        out_shape=jax.ShapeDtypeStruct((N_NODES, HID), f32),
    )(nodes, parts_m, parts_e, Wu1[:D_FEAT], Wu1[D_FEAT:],
      bu1.reshape(1, HID), Wu2, bu2.reshape(1, HID))

    return out


# speedup vs baseline: 1.6526x; 1.6526x over previous
import functools

import jax
import jax.numpy as jnp
from jax import lax
from jax.experimental import pallas as pl
from jax.experimental.pallas import tpu as pltpu
from jax.experimental.pallas import tpu_sc as plsc

N_NODES = 10000
N_EDGES = 320000
D_FEAT = 128
D_EDGE = 16
HID = 128

NC, NS = 2, 16
NW = NC * NS
EPW = N_EDGES // NW
GCH = 80
NCH = EPW // GCH
N_PAD = 10240
NPS = N_PAD // NS
L = 16

_SC_PARAMS = pltpu.CompilerParams(use_tc_tiling_on_sc=False)
_SC_PARAMS_NL = pltpu.CompilerParams(use_tc_tiling_on_sc=False,
                                     needs_layout_passes=False)


@functools.cache
def _mesh():
    return plsc.VectorSubcoreMesh(
        core_axis_name="c", subcore_axis_name="s",
        num_cores=NC, num_subcores=NS)


def _prelude_body(nodes_ref, wsq_ref, p_ref, q_ref):
    pq = jnp.dot(nodes_ref[...], wsq_ref[...],
                 preferred_element_type=jnp.float32)
    p_ref[...] = pq[:, :HID]
    q_ref[...] = pq[:, HID:]


def _msg_body(e_ref, sr_ref, w1e_ref, b1_ref, w2_ref, b2_ref,
              wg1_ref, bg1_ref, wg2p_ref, bg2_ref, m_ref, x_ref):
    h = sr_ref[...] + jnp.dot(e_ref[...], w1e_ref[...],
                              preferred_element_type=jnp.float32) + b1_ref[...]
    h = jnp.maximum(h, 0.0)
    msgs = jnp.dot(h, w2_ref[...], preferred_element_type=jnp.float32) + b2_ref[...]
    g = jnp.maximum(jnp.dot(msgs, wg1_ref[...],
                            preferred_element_type=jnp.float32) + bg1_ref[...], 0.0)
    gate = jnp.dot(g, wg2p_ref[...],
                   preferred_element_type=jnp.float32) + bg2_ref[...]
    ex16 = jnp.exp(gate)
    m_ref[...] = ex16[:, 0:1] * msgs
    x_ref[...] = ex16[:, 0:1].reshape(x_ref.shape)


def _upd_body(nodes_ref, am_ref, ae_ref, ones_ref, wu1a_ref, wu1b_ref,
              bu1_ref, wu2_ref, bu2_ref, out_ref):
    am = am_ref[...]
    num = am[0] + am[1]
    den = jnp.dot(ae_ref[...], ones_ref[...],
                  preferred_element_type=jnp.float32)
    aggr = jnp.where(den > 0.0, num / den, 0.0)
    u = (jnp.dot(nodes_ref[...], wu1a_ref[...], preferred_element_type=jnp.float32)
         + jnp.dot(aggr, wu1b_ref[...], preferred_element_type=jnp.float32)
         + bu1_ref[...])
    u = jnp.maximum(u, 0.0)
    out_ref[...] = jnp.dot(u, wu2_ref[...],
                           preferred_element_type=jnp.float32) + bu2_ref[...]


@functools.cache
def _gather_kernel():
    return pl.kernel(
        _gather_body,
        out_type=jax.ShapeDtypeStruct((N_EDGES, HID), jnp.float32),
        mesh=_mesh(),
        compiler_params=_SC_PARAMS,
        scratch_types=(
            [pltpu.VMEM((NCH, GCH), jnp.int32)] * 2
            + [pltpu.VMEM((GCH, HID), jnp.float32)] * 4
            + [pltpu.SemaphoreType.DMA] * 4
        ),
    )


def _gather_body(p_hbm, q_hbm, s3_hbm, r3_hbm, out_hbm,
                 sidx2, ridx2, pb0, qb0, pb1, qb1, gs0, gs1, ws0, ws1):
    wid = lax.axis_index("s") * NC + lax.axis_index("c")
    base0 = wid * EPW
    pltpu.sync_copy(s3_hbm.at[wid], sidx2)
    pltpu.sync_copy(r3_hbm.at[wid], ridx2)

    def issue(ci, pb, qb, gs):
        pltpu.async_copy(p_hbm.at[sidx2.at[ci]], pb, gs)
        pltpu.async_copy(q_hbm.at[ridx2.at[ci]], qb, gs)

    def gwait(pb, qb, gs):
        pltpu.make_async_copy(p_hbm.at[sidx2.at[0]], pb, gs).wait()
        pltpu.make_async_copy(q_hbm.at[ridx2.at[0]], qb, gs).wait()

    def add_rows(pb, qb):
        @pl.loop(0, GCH)
        def _r(e):
            for c in range(0, HID, L):
                pb[e, pl.ds(c, L)] = pb[e, pl.ds(c, L)] + qb[e, pl.ds(c, L)]

    def wstart(ci, pb, ws):
        pltpu.async_copy(pb, out_hbm.at[pl.ds(base0 + ci * GCH, GCH)], ws)

    def wwait(pb, ws):
        pltpu.make_async_copy(pb, out_hbm.at[pl.ds(base0, GCH)], ws).wait()

    issue(0, pb0, qb0, gs0)
    issue(1, pb1, qb1, gs1)

    @pl.loop(0, (NCH - 1) // 2)
    def _k(k):
        gwait(pb0, qb0, gs0)
        add_rows(pb0, qb0)
        wstart(2 * k, pb0, ws0)
        wwait(pb0, ws0)
        issue(2 * k + 2, pb0, qb0, gs0)

        gwait(pb1, qb1, gs1)
        add_rows(pb1, qb1)
        wstart(2 * k + 1, pb1, ws1)
        wwait(pb1, ws1)
        issue(jnp.minimum(2 * k + 3, NCH - 1), pb1, qb1, gs1)

    gwait(pb0, qb0, gs0)
    add_rows(pb0, qb0)
    wstart(NCH - 1, pb0, ws0)
    wwait(pb0, ws0)
    gwait(pb1, qb1, gs1)


@functools.cache
def _scatter_kernel():
    return pl.kernel(
        _scatter_body,
        out_type=[
            jax.ShapeDtypeStruct((NC, N_PAD, HID), jnp.float32),
            jax.ShapeDtypeStruct((NW, N_PAD), jnp.float32),
        ],
        mesh=_mesh(),
        compiler_params=_SC_PARAMS_NL,
        scratch_types=[
            pltpu.VMEM((EPW,), jnp.int32),
            pltpu.VMEM((GCH, HID), jnp.float32),
            pltpu.VMEM((GCH,), jnp.float32),
            pltpu.VMEM((GCH, HID), jnp.float32),
            pltpu.VMEM((GCH,), jnp.float32),
            pltpu.VMEM((N_PAD,), jnp.float32),
            pltpu.VMEM_SHARED((N_PAD, HID), jnp.float32),
            pltpu.SemaphoreType.DMA,
            pltpu.SemaphoreType.DMA,
            pltpu.SemaphoreType.DMA,
            pltpu.SemaphoreType.DMA,
        ],
    )


def _scatter_body(exm_hbm, ex_hbm, r2_hbm, z_hbm, om_hbm, oe_hbm,
                  ridx, mb0, xb0, mb1, xb1, den, accm,
                  ls0, ls1, ss0, ss1):
    cid = lax.axis_index("c")
    sid = lax.axis_index("s")
    wid = sid * NC + cid
    base0 = wid * EPW

    @pl.loop(0, N_PAD // L)
    def _zd(i):
        den[pl.ds(i * L, L)] = jnp.zeros((L,), jnp.float32)

    pltpu.sync_copy(z_hbm, mb0)

    @pl.loop(0, NPS // GCH)
    def _zc(j):
        pltpu.sync_copy(mb0, accm.at[pl.ds(sid * NPS + j * GCH, GCH)])

    pltpu.sync_copy(r2_hbm.at[wid], ridx)
    plsc.subcore_barrier()

    def lissue(ci, mb, xb, ls):
        pltpu.async_copy(exm_hbm.at[pl.ds(base0 + ci * GCH, GCH)], mb, ls)
        pltpu.async_copy(ex_hbm.at[wid, pl.ds(ci * GCH, GCH)], xb, ls)

    def lwait(mb, xb, ls):
        pltpu.make_async_copy(exm_hbm.at[pl.ds(base0, GCH)], mb, ls).wait()
        pltpu.make_async_copy(ex_hbm.at[wid, pl.ds(0, GCH)], xb, ls).wait()

    def den_scatter(ci, xb):
        @pl.loop(0, GCH // L)
        def _j(j):
            idxv = ridx[pl.ds(ci * GCH + j * L, L)]
            exv = xb[pl.ds(j * L, L)]
            plsc.addupdate_scatter(den, [idxv], exv)

    def sstart(ci, mb, ss):
        pltpu.async_copy(mb, accm.at[ridx.at[pl.ds(ci * GCH, GCH)]], ss,
                         add=True)

    def swait(ci, mb, ss):
        pltpu.make_async_copy(mb, accm.at[ridx.at[pl.ds(ci * GCH, GCH)]],
                              ss).wait()

    lissue(0, mb0, xb0, ls0)
    lissue(1, mb1, xb1, ls1)

    @pl.loop(0, (NCH - 1) // 2)
    def _k(k):
        lwait(mb0, xb0, ls0)
        sstart(2 * k, mb0, ss0)
        den_scatter(2 * k, xb0)
        swait(2 * k, mb0, ss0)
        lissue(2 * k + 2, mb0, xb0, ls0)

        lwait(mb1, xb1, ls1)
        sstart(2 * k + 1, mb1, ss1)
        den_scatter(2 * k + 1, xb1)
        swait(2 * k + 1, mb1, ss1)
        lissue(jnp.minimum(2 * k + 3, NCH - 1), mb1, xb1, ls1)

    lwait(mb0, xb0, ls0)
    sstart(NCH - 1, mb0, ss0)
    den_scatter(NCH - 1, xb0)
    swait(NCH - 1, mb0, ss0)
    lwait(mb1, xb1, ls1)

    plsc.subcore_barrier()
    pltpu.sync_copy(accm.at[pl.ds(sid * NPS, NPS)],
                    om_hbm.at[cid, pl.ds(sid * NPS, NPS)])
    pltpu.sync_copy(den, oe_hbm.at[wid])


def kernel(nodes, edges, W1, b1, W2, b2, Wg1, bg1, Wg2, bg2,
           Wu1, bu1, Wu2, bu2, senders, receivers):
    f32 = jnp.float32
    s3 = senders.astype(jnp.int32).reshape(NW, NCH, GCH)
    r3 = receivers.astype(jnp.int32).reshape(NW, NCH, GCH)
    r2 = receivers.astype(jnp.int32).reshape(NW, EPW)

    wsq = W1[D_EDGE:, :]
    wsq = jnp.concatenate([wsq[:D_FEAT], wsq[D_FEAT:]], axis=1)
    w1e = W1[:D_EDGE, :]
    wg2p = jnp.tile(Wg2, (1, L))

    NB = 2000
    p, q = pl.pallas_call(
        _prelude_body,
        grid=(N_NODES // NB,),
        in_specs=[
            pl.BlockSpec((NB, D_FEAT), lambda i: (i, 0)),
            pl.BlockSpec((D_FEAT, 2 * HID), lambda i: (0, 0)),
        ],
        out_specs=[
            pl.BlockSpec((NB, HID), lambda i: (i, 0)),
            pl.BlockSpec((NB, HID), lambda i: (i, 0)),
        ],
        out_shape=[
            jax.ShapeDtypeStruct((N_NODES, HID), f32),
            jax.ShapeDtypeStruct((N_NODES, HID), f32),
        ],
    )(nodes, wsq)

    sr = _gather_kernel()(p, q, s3, r3)

    EB = 2560
    exm, ex = pl.pallas_call(
        _msg_body,
        grid=(N_EDGES // EB,),
        in_specs=[
            pl.BlockSpec((EB, D_EDGE), lambda i: (i, 0)),
            pl.BlockSpec((EB, HID), lambda i: (i, 0)),
            pl.BlockSpec((D_EDGE, HID), lambda i: (0, 0)),
            pl.BlockSpec((1, HID), lambda i: (0, 0)),
            pl.BlockSpec((HID, HID), lambda i: (0, 0)),
            pl.BlockSpec((1, HID), lambda i: (0, 0)),
            pl.BlockSpec((HID, HID), lambda i: (0, 0)),
            pl.BlockSpec((1, HID), lambda i: (0, 0)),
            pl.BlockSpec((HID, L), lambda i: (0, 0)),
            pl.BlockSpec((1, 1), lambda i: (0, 0)),
        ],
        out_specs=[
            pl.BlockSpec((EB, HID), lambda i: (i, 0)),
            pl.BlockSpec((1, EB // 128, 128), lambda i: (i, 0, 0)),
        ],
        out_shape=[
            jax.ShapeDtypeStruct((N_EDGES, HID), f32),
            jax.ShapeDtypeStruct((N_EDGES // EB, EB // 128, 128), f32),
        ],
    )(edges, sr, w1e, b1.reshape(1, HID), W2, b2.reshape(1, HID),
      Wg1, bg1.reshape(1, HID), wg2p, bg2.reshape(1, 1))

    zeros = jnp.zeros((GCH, HID), f32)
    parts_m, parts_e = _scatter_kernel()(exm, ex.reshape(NW, EPW), r2, zeros)

    UB = 2000
    out = pl.pallas_call(
        _upd_body,
        grid=(N_NODES // UB,),
        in_specs=[
            pl.BlockSpec((UB, D_FEAT), lambda i: (i, 0)),
            pl.BlockSpec((NC, UB, HID), lambda i: (0, i, 0)),
            pl.BlockSpec((UB, NW), lambda i: (i, 0)),
            pl.BlockSpec((NW, 1), lambda i: (0, 0)),
            pl.BlockSpec((D_FEAT, HID), lambda i: (0, 0)),
            pl.BlockSpec((HID, HID), lambda i: (0, 0)),
            pl.BlockSpec((1, HID), lambda i: (0, 0)),
            pl.BlockSpec((HID, HID), lambda i: (0, 0)),
            pl.BlockSpec((1, HID), lambda i: (0, 0)),
        ],
        out_specs=pl.BlockSpec((UB, HID), lambda i: (i, 0)),
        out_shape=jax.ShapeDtypeStruct((N_NODES, HID), f32),
    )(nodes, parts_m, parts_e.T, jnp.ones((NW, 1), f32), Wu1[:D_FEAT],
      Wu1[D_FEAT:], bu1.reshape(1, HID), Wu2, bu2.reshape(1, HID))

    return out
